# Initial kernel scaffold; baseline (speedup 1.0000x reference)
import jax
import jax.numpy as jnp
from jax.experimental import pallas as pl


def kernel(x, Wqkv, bqkv, Wlepe, blepe, Wout, bout):
    raise NotImplementedError("write your pallas kernel here")


# speedup vs baseline: 3.5285x; 3.5285x over previous
import jax
import jax.numpy as jnp
from jax.experimental import pallas as pl
from jax.experimental.pallas import tpu as pltpu

_DIM = 384
_HEADS = 12
_HD = 32
_NW = 7
_TOPK = 4
_RS = 8
_B = 4
_H = 56
_W = 56


def _stage1(x_ref, w_ref, b_ref, qkv_ref, pq_ref, pk_ref):
    xr = x_ref[0].reshape(_RS * _W, _DIM)
    y = jnp.dot(xr, w_ref[...], preferred_element_type=jnp.float32) + b_ref[...]
    qkv_ref[0] = y.reshape(_RS, _W, 3 * _DIM)
    pm = y.reshape(_RS, _NW, _RS, 3 * _DIM).mean(axis=(0, 2))
    pq_ref[0, 0] = pm[:, :_DIM]
    pk_ref[0, 0] = pm[:, _DIM:2 * _DIM]


def _route(pq_ref, pk_ref, idx_ref):
    nr = _NW * _NW
    qm = pq_ref[0].reshape(nr, _DIM)
    km = pk_ref[0].reshape(nr, _DIM)
    a = jax.lax.dot_general(qm, km, (((1,), (1,)), ((), ())),
                            preferred_element_type=jnp.float32)
    iota = jax.lax.broadcasted_iota(jnp.int32, (nr, nr), 1)
    cols = []
    for _ in range(_TOPK):
        m = jnp.max(a, axis=1, keepdims=True)
        sel = jnp.where(a >= m, iota, nr)
        it = jnp.min(sel, axis=1, keepdims=True)
        cols.append(it)
        a = jnp.where(iota == it, -jnp.inf, a)
    idx4 = jnp.concatenate(cols, axis=1)
    idx_ref[0] = jnp.concatenate(
        [idx4, jnp.zeros((nr, 128 - _TOPK), jnp.int32)], axis=1)


def _attn(idx_ref, q_ref, k0, k1, k2, k3, v0, v1, v2, v3, o_ref):
    scale = _DIM ** -0.5
    q = q_ref[0].reshape(64, _DIM) * scale
    K = jnp.concatenate([r[0].reshape(64, _DIM) for r in (k0, k1, k2, k3)],
                        axis=0)
    V = jnp.concatenate([r[0].reshape(64, _DIM) for r in (v0, v1, v2, v3)],
                        axis=0)
    q3 = q.reshape(64, _HEADS, _HD)
    k3_ = K.reshape(4 * 64, _HEADS, _HD)
    v3_ = V.reshape(4 * 64, _HEADS, _HD)
    logits = jax.lax.dot_general(
        q3, k3_, (((2,), (2,)), ((1,), (1,))),
        preferred_element_type=jnp.float32)
    m = jnp.max(logits, axis=-1, keepdims=True)
    e = jnp.exp(logits - m)
    p = e / jnp.sum(e, axis=-1, keepdims=True)
    o = jax.lax.dot_general(
        p, v3_, (((2,), (0,)), ((0,), (1,))),
        preferred_element_type=jnp.float32)
    o_ref[0] = o.transpose(1, 0, 2).reshape(_RS, _RS, _DIM)


def _stage4(at_ref, vp_ref, wl_ref, wo_ref, bo_ref, o_ref):
    i = pl.program_id(1)
    acc = at_ref[0]
    vp = vp_ref[0, pl.ds(i * _RS, _RS + 4), :, :]
    for di in range(5):
        for dj in range(5):
            coef = wl_ref[0, di * 5 + dj]
            acc = acc + coef[None, None, :] * vp[di:di + _RS, dj:dj + _W, :]
    y = jnp.dot(acc.reshape(_RS * _W, _DIM), wo_ref[...],
                preferred_element_type=jnp.float32) + bo_ref[...]
    o_ref[0] = y.reshape(_RS, _W, _DIM)


def kernel(x, Wqkv, bqkv, Wlepe, blepe, Wout, bout):
    f32 = jnp.float32
    nr = _NW * _NW
    x_n = x.transpose(0, 2, 3, 1)
    W2 = Wqkv.reshape(3 * _DIM, _DIM).T
    b2 = bqkv.reshape(1, 3 * _DIM)

    qkv, pq, pk = pl.pallas_call(
        _stage1,
        grid=(_B, _NW),
        in_specs=[
            pl.BlockSpec((1, _RS, _W, _DIM), lambda b, i: (b, i, 0, 0)),
            pl.BlockSpec((_DIM, 3 * _DIM), lambda b, i: (0, 0)),
            pl.BlockSpec((1, 3 * _DIM), lambda b, i: (0, 0)),
        ],
        out_specs=[
            pl.BlockSpec((1, _RS, _W, 3 * _DIM), lambda b, i: (b, i, 0, 0)),
            pl.BlockSpec((1, 1, _NW, _DIM), lambda b, i: (b, i, 0, 0)),
            pl.BlockSpec((1, 1, _NW, _DIM), lambda b, i: (b, i, 0, 0)),
        ],
        out_shape=[
            jax.ShapeDtypeStruct((_B, _H, _W, 3 * _DIM), f32),
            jax.ShapeDtypeStruct((_B, _NW, _NW, _DIM), f32),
            jax.ShapeDtypeStruct((_B, _NW, _NW, _DIM), f32),
        ],
    )(x_n, W2, b2)

    idx_pad = pl.pallas_call(
        _route,
        grid=(_B,),
        in_specs=[
            pl.BlockSpec((1, _NW, _NW, _DIM), lambda b: (b, 0, 0, 0)),
            pl.BlockSpec((1, _NW, _NW, _DIM), lambda b: (b, 0, 0, 0)),
        ],
        out_specs=pl.BlockSpec((1, nr, 128), lambda b: (b, 0, 0)),
        out_shape=jax.ShapeDtypeStruct((_B, nr, 128), jnp.int32),
    )(pq, pk)
    idx = idx_pad[:, :, :_TOPK]

    def q_map(b, r, idx_ref):
        return (b, r // _NW, r % _NW, 0)

    def kv_map(t, cblk):
        def m(b, r, idx_ref):
            tr = idx_ref[b, r, t]
            return (b, tr // _NW, tr % _NW, cblk)
        return m

    in_specs = [pl.BlockSpec((1, _RS, _RS, _DIM), q_map)]
    for t in range(_TOPK):
        in_specs.append(pl.BlockSpec((1, _RS, _RS, _DIM), kv_map(t, 1)))
    for t in range(_TOPK):
        in_specs.append(pl.BlockSpec((1, _RS, _RS, _DIM), kv_map(t, 2)))

    grid_spec = pltpu.PrefetchScalarGridSpec(
        num_scalar_prefetch=1,
        grid=(_B, nr),
        in_specs=in_specs,
        out_specs=pl.BlockSpec((1, _RS, _RS, _DIM), q_map),
    )
    attn = pl.pallas_call(
        _attn,
        grid_spec=grid_spec,
        out_shape=jax.ShapeDtypeStruct((_B, _H, _W, _DIM), f32),
    )(idx, *([qkv] * 9))

    v_pad = jnp.pad(qkv[:, :, :, 2 * _DIM:], ((0, 0), (2, 2), (2, 2), (0, 0)))
    wl = Wlepe.reshape(1, _DIM, 25).transpose(0, 2, 1)
    Wo = Wout.reshape(_DIM, _DIM).T
    bo = bout.reshape(1, _DIM)

    out_n = pl.pallas_call(
        _stage4,
        grid=(_B, _NW),
        in_specs=[
            pl.BlockSpec((1, _RS, _W, _DIM), lambda b, i: (b, i, 0, 0)),
            pl.BlockSpec((1, _H + 4, _W + 4, _DIM), lambda b, i: (b, 0, 0, 0)),
            pl.BlockSpec((1, 25, _DIM), lambda b, i: (0, 0, 0)),
            pl.BlockSpec((_DIM, _DIM), lambda b, i: (0, 0)),
            pl.BlockSpec((1, _DIM), lambda b, i: (0, 0)),
        ],
        out_specs=pl.BlockSpec((1, _RS, _W, _DIM), lambda b, i: (b, i, 0, 0)),
        out_shape=jax.ShapeDtypeStruct((_B, _H, _W, _DIM), f32),
    )(attn, v_pad, wl, Wo, bo)
    return out_n.transpose(0, 3, 1, 2)
